# baseline (speedup 1.0000x reference)
import functools

import jax
import jax.numpy as jnp
from jax import lax
from jax.experimental import pallas as pl
from jax.experimental.pallas import tpu as pltpu
from jax.experimental.pallas import tpu_sc as plsc

IN_N = 2048
M_TOTAL = 16384
NC, NS = 2, 16
NW = NC * NS
ROWS_PER_W = M_TOTAL // NW
CH = 16
NCHUNK = ROWS_PER_W // CH


def _sc_copy(x_hbm, o_hbm, buf_a, buf_b, rsem, wsem):
    wid = lax.axis_index("s") * NC + lax.axis_index("c")
    base = wid * ROWS_PER_W
    bufs = (buf_a, buf_b)

    def rd(c, b):
        return pltpu.make_async_copy(
            x_hbm.at[pl.ds(base + c * CH, CH), :], bufs[b], rsem.at[b])

    def wr(c, b):
        return pltpu.make_async_copy(
            bufs[b], o_hbm.at[pl.ds(base + c * CH, CH), :], wsem.at[b])

    rd(0, 0).start()
    rd(1, 1).start()
    for c in range(NCHUNK):
        b = c % 2
        rd(c, b).wait()
        wr(c, b).start()
        if c + 2 < NCHUNK:
            wr(c, b).wait()
            rd(c + 2, b).start()
    wr(NCHUNK - 2, NCHUNK % 2).wait()
    wr(NCHUNK - 1, (NCHUNK - 1) % 2).wait()


def kernel(input):
    x = input.reshape(-1, IN_N)
    mesh = plsc.VectorSubcoreMesh(core_axis_name="c", subcore_axis_name="s")
    f = functools.partial(
        pl.kernel,
        mesh=mesh,
        out_type=jax.ShapeDtypeStruct((M_TOTAL, IN_N), jnp.float32),
        scratch_types=[
            pltpu.VMEM((CH, IN_N), jnp.float32),
            pltpu.VMEM((CH, IN_N), jnp.float32),
            pltpu.SemaphoreType.DMA((2,)),
            pltpu.SemaphoreType.DMA((2,)),
        ],
    )(_sc_copy)
    return f(x)
